# Initial kernel scaffold; baseline (speedup 1.0000x reference)
import jax
import jax.numpy as jnp
from jax.experimental import pallas as pl


def kernel(tokens, table, ln_gamma, ln_beta):
    raise NotImplementedError("write your pallas kernel here")


# speedup vs baseline: 3.2880x; 3.2880x over previous
import functools
import math

import jax
import jax.numpy as jnp
from jax import lax
from jax.experimental import pallas as pl

HIDDEN = 1024
VOCAB = 30
PAD_IDX = 1
SEQ_BLOCK = 256


def _body(tok_ref, table_ref, gamma_ref, beta_ref, out_ref):
    i = pl.program_id(0)
    batch, bs = tok_ref.shape
    hidden = table_ref.shape[1]
    half = hidden // 2

    tok = tok_ref[...]

    seq_len = pl.num_programs(0) * bs
    s_idx = i * bs + lax.broadcasted_iota(jnp.int32, (bs, 1), 0)
    pos = (seq_len - 1 - s_idx).astype(jnp.float32)
    j = lax.broadcasted_iota(jnp.int32, (1, half), 1).astype(jnp.float32)
    invf = jnp.exp(j * (-math.log(10000.0) / half))
    angle = pos * invf
    pe = jnp.concatenate([jnp.sin(angle), jnp.cos(angle)], axis=-1)

    vocab_iota = lax.broadcasted_iota(jnp.int32, (VOCAB, 1), 0)
    pad_col = (vocab_iota == PAD_IDX).astype(jnp.float32)
    gamma = gamma_ref[...][None, :]
    beta = beta_ref[...][None, :]
    table = table_ref[...]

    for b in range(batch):
        tok_b = tok[b : b + 1, :]
        onehot = (vocab_iota == tok_b).astype(jnp.float32)
        x = lax.dot_general(
            onehot, table, (((0,), (0,)), ((), ())),
            preferred_element_type=jnp.float32,
        )
        x = x + pe
        mu = jnp.mean(x, axis=-1, keepdims=True)
        xc = x - mu
        var = jnp.mean(xc * xc, axis=-1, keepdims=True)
        y = xc * lax.rsqrt(var + 1e-12) * gamma + beta
        is_pad = lax.dot_general(
            onehot, pad_col, (((0,), (0,)), ((), ())),
            preferred_element_type=jnp.float32,
        )
        out_ref[b, :, :] = y * (1.0 - is_pad)


@jax.jit
def kernel(tokens, table, ln_gamma, ln_beta):
    tokens = tokens.astype(jnp.int32)
    batch, seq = tokens.shape
    hidden = table.shape[1]
    grid = seq // SEQ_BLOCK

    return pl.pallas_call(
        _body,
        grid=(grid,),
        in_specs=[
            pl.BlockSpec((batch, SEQ_BLOCK), lambda i: (0, i)),
            pl.BlockSpec((VOCAB, hidden), lambda i: (0, 0)),
            pl.BlockSpec((hidden,), lambda i: (0,)),
            pl.BlockSpec((hidden,), lambda i: (0,)),
        ],
        out_specs=pl.BlockSpec((batch, SEQ_BLOCK, hidden), lambda i: (0, i, 0)),
        out_shape=jax.ShapeDtypeStruct((batch, seq, hidden), jnp.float32),
    )(tokens, table, ln_gamma, ln_beta)
